# baseline (speedup 1.0000x reference)
import functools

import jax
import jax.numpy as jnp
from jax import lax
from jax.experimental import pallas as pl
from jax.experimental.pallas import tpu as pltpu
from jax.experimental.pallas import tpu_sc as plsc

ROWS = 128
COLS = 32768
LANES = 16
NUM_CORES = 2
NUM_SUBCORES = 16
NUM_WORKERS = NUM_CORES * NUM_SUBCORES
SC_ROWS_PER_WORKER = 1
SC_ROWS = NUM_WORKERS * SC_ROWS_PER_WORKER
TC_ROWS = ROWS - SC_ROWS
VECS = COLS // LANES
UNROLL = 16
NACC = 4
TC_BLOCK_ROWS = 16


def _row_argmin(buf, lane_iota):
    def body(i, carry):
        minvs, minis = carry
        minvs = list(minvs)
        minis = list(minis)
        base = i * (LANES * UNROLL)
        for u in range(UNROLL):
            k = u % NACC
            off = base + u * LANES
            v = buf[pl.ds(off, LANES)]
            idxv = lane_iota + off
            pred = v < minvs[k]
            minvs[k] = jnp.where(pred, v, minvs[k])
            minis[k] = jnp.where(pred, idxv, minis[k])
        return tuple(minvs), tuple(minis)

    minv0 = jnp.full((LANES,), jnp.inf, jnp.float32)
    mini0 = jnp.zeros((LANES,), jnp.int32)
    minvs, minis = lax.fori_loop(
        0, VECS // UNROLL, body, ((minv0,) * NACC, (mini0,) * NACC)
    )
    minv, mini = minvs[0], minis[0]
    for k in range(1, NACC):
        pred = (minvs[k] < minv) | ((minvs[k] == minv) & (minis[k] < mini))
        minv = jnp.where(pred, minvs[k], minv)
        mini = jnp.where(pred, minis[k], mini)
    m = jnp.min(minv)
    cand = jnp.where(minv == m, mini, jnp.int32(COLS))
    return jnp.min(cand)


@functools.partial(
    pl.kernel,
    out_type=jax.ShapeDtypeStruct((SC_ROWS, LANES), jnp.int32),
    mesh=plsc.VectorSubcoreMesh(
        core_axis_name="c",
        subcore_axis_name="s",
        num_cores=NUM_CORES,
        num_subcores=NUM_SUBCORES,
    ),
    scratch_types=[
        pltpu.VMEM((COLS,), jnp.float32),
        pltpu.VMEM((COLS,), jnp.float32),
        pltpu.VMEM((LANES,), jnp.int32),
        pltpu.SemaphoreType.DMA,
        pltpu.SemaphoreType.DMA,
    ],
    compiler_params=pltpu.CompilerParams(needs_layout_passes=False),
)
def _argmin_sc(x_hbm, out_hbm, buf0, buf1, res_ref, sem0, sem1):
    wid = lax.axis_index("s") * NUM_CORES + lax.axis_index("c")
    base = wid * SC_ROWS_PER_WORKER
    lane_iota = lax.iota(jnp.int32, LANES)
    bufs = (buf0, buf1)
    sems = (sem0, sem1)
    copies = [None] * SC_ROWS_PER_WORKER
    copies[0] = pltpu.async_copy(x_hbm.at[base], buf0, sem0)
    res = jnp.zeros((LANES,), jnp.int32)
    for j in range(SC_ROWS_PER_WORKER):
        copies[j].wait()
        if j + 1 < SC_ROWS_PER_WORKER:
            copies[j + 1] = pltpu.async_copy(
                x_hbm.at[base + j + 1], bufs[(j + 1) % 2], sems[(j + 1) % 2]
            )
        val = _row_argmin(bufs[j % 2], lane_iota)
        res_ref[...] = jnp.where(lane_iota == 0, val, res)
        pltpu.sync_copy(res_ref, out_hbm.at[base + j])


def _argmin_tc_body(x_ref, out_ref):
    xb = x_ref[...]
    rm = jnp.min(xb, axis=1, keepdims=True)
    idx = lax.broadcasted_iota(jnp.int32, (TC_BLOCK_ROWS, COLS), 1)
    cand = jnp.where(xb == rm, idx, jnp.int32(COLS))
    out_ref[...] = jnp.min(cand, axis=1, keepdims=True)


_argmin_tc = pl.pallas_call(
    _argmin_tc_body,
    grid=(TC_ROWS // TC_BLOCK_ROWS,),
    in_specs=[
        pl.BlockSpec(
            (TC_BLOCK_ROWS, COLS), lambda i: (i + SC_ROWS // TC_BLOCK_ROWS, 0)
        )
    ],
    out_specs=pl.BlockSpec((TC_BLOCK_ROWS, 1), lambda i: (i, 0)),
    out_shape=jax.ShapeDtypeStruct((TC_ROWS, 1), jnp.int32),
)


def _combine_body(sc_ref, tc_ref, kd_ref, flat_ref):
    sc_col = jnp.squeeze(sc_ref[...][:, :1], axis=1)
    tc_flat = jnp.squeeze(tc_ref[...], axis=1)
    flat = jnp.concatenate([sc_col, tc_flat])
    flat_ref[...] = flat
    kd_ref[...] = lax.broadcast_in_dim(flat, (ROWS, 1), (0,))


_combine = pl.pallas_call(
    _combine_body,
    out_shape=(
        jax.ShapeDtypeStruct((ROWS, 1), jnp.int32),
        jax.ShapeDtypeStruct((ROWS,), jnp.int32),
    ),
)


def kernel(x):
    tc_out = _argmin_tc(x)
    sc_pad = _argmin_sc(x)
    return _combine(sc_pad, tc_out)


# speedup vs baseline: 1.0117x; 1.0117x over previous
import functools

import jax
import jax.numpy as jnp
from jax import lax
from jax.experimental import pallas as pl
from jax.experimental.pallas import tpu as pltpu
from jax.experimental.pallas import tpu_sc as plsc

ROWS = 128
COLS = 32768
LANES = 16
NUM_CORES = 2
NUM_SUBCORES = 16
NUM_WORKERS = NUM_CORES * NUM_SUBCORES
SC_ROWS_PER_WORKER = 1
SC_ROWS = NUM_WORKERS * SC_ROWS_PER_WORKER
TC_ROWS = ROWS - SC_ROWS
VECS = COLS // LANES
UNROLL = 16
NACC = 4
TC_BLOCK_ROWS = 16


def _row_argmin(buf, lane_iota):
    def body(i, carry):
        minvs, minis = carry
        minvs = list(minvs)
        minis = list(minis)
        base = i * (LANES * UNROLL)
        for u in range(UNROLL):
            k = u % NACC
            off = base + u * LANES
            v = buf[pl.ds(off, LANES)]
            idxv = lane_iota + off
            pred = v < minvs[k]
            minvs[k] = jnp.where(pred, v, minvs[k])
            minis[k] = jnp.where(pred, idxv, minis[k])
        return tuple(minvs), tuple(minis)

    minv0 = jnp.full((LANES,), jnp.inf, jnp.float32)
    mini0 = jnp.zeros((LANES,), jnp.int32)
    minvs, minis = lax.fori_loop(
        0, VECS // UNROLL, body, ((minv0,) * NACC, (mini0,) * NACC)
    )
    minv, mini = minvs[0], minis[0]
    for k in range(1, NACC):
        pred = (minvs[k] < minv) | ((minvs[k] == minv) & (minis[k] < mini))
        minv = jnp.where(pred, minvs[k], minv)
        mini = jnp.where(pred, minis[k], mini)
    m = jnp.min(minv)
    cand = jnp.where(minv == m, mini, jnp.int32(COLS))
    return jnp.min(cand)


@functools.partial(
    pl.kernel,
    out_type=jax.ShapeDtypeStruct((SC_ROWS, LANES), jnp.int32),
    mesh=plsc.VectorSubcoreMesh(
        core_axis_name="c",
        subcore_axis_name="s",
        num_cores=NUM_CORES,
        num_subcores=NUM_SUBCORES,
    ),
    scratch_types=[
        pltpu.VMEM((COLS,), jnp.float32),
        pltpu.VMEM((COLS,), jnp.float32),
        pltpu.VMEM((LANES,), jnp.int32),
        pltpu.SemaphoreType.DMA,
        pltpu.SemaphoreType.DMA,
    ],
    compiler_params=pltpu.CompilerParams(needs_layout_passes=False),
)
def _argmin_sc(x_hbm, out_hbm, buf0, buf1, res_ref, sem0, sem1):
    wid = lax.axis_index("s") * NUM_CORES + lax.axis_index("c")
    base = wid * SC_ROWS_PER_WORKER
    lane_iota = lax.iota(jnp.int32, LANES)
    bufs = (buf0, buf1)
    sems = (sem0, sem1)
    copies = [None] * SC_ROWS_PER_WORKER
    copies[0] = pltpu.async_copy(x_hbm.at[base], buf0, sem0)
    res = jnp.zeros((LANES,), jnp.int32)
    for j in range(SC_ROWS_PER_WORKER):
        copies[j].wait()
        if j + 1 < SC_ROWS_PER_WORKER:
            copies[j + 1] = pltpu.async_copy(
                x_hbm.at[base + j + 1], bufs[(j + 1) % 2], sems[(j + 1) % 2]
            )
        val = _row_argmin(bufs[j % 2], lane_iota)
        res_ref[...] = jnp.where(lane_iota == 0, val, res)
        pltpu.sync_copy(res_ref, out_hbm.at[base + j])


def _argmin_tc_body(x_ref, out_ref):
    xb = x_ref[...]
    rm = jnp.min(xb, axis=1, keepdims=True)
    idx = lax.broadcasted_iota(jnp.int32, (TC_BLOCK_ROWS, COLS), 1)
    cand = jnp.where(xb == rm, idx, jnp.int32(COLS))
    out_ref[...] = jnp.min(cand, axis=1).reshape(1, 1, TC_BLOCK_ROWS)


_argmin_tc = pl.pallas_call(
    _argmin_tc_body,
    grid=(TC_ROWS // TC_BLOCK_ROWS,),
    in_specs=[
        pl.BlockSpec(
            (TC_BLOCK_ROWS, COLS), lambda i: (i + SC_ROWS // TC_BLOCK_ROWS, 0)
        )
    ],
    out_specs=pl.BlockSpec((1, 1, TC_BLOCK_ROWS), lambda i: (i, 0, 0)),
    out_shape=jax.ShapeDtypeStruct(
        (TC_ROWS // TC_BLOCK_ROWS, 1, TC_BLOCK_ROWS), jnp.int32
    ),
)


def kernel(x):
    tc_out = _argmin_tc(x)
    sc_pad = _argmin_sc(x)
    flat = jnp.concatenate([sc_pad[:, 0], tc_out.reshape(TC_ROWS)])
    return (flat.reshape(ROWS, 1), flat)
